# baseline (speedup 1.0000x reference)
import jax
import jax.numpy as jnp
from jax import lax
from jax.experimental import pallas as pl
from jax.experimental.pallas import tpu as pltpu
from jax.experimental.pallas import tpu_sc as plsc

NB_TASKS = 1000
D = 128
BATCH = 16384

SC_NC = 1
SC_NS = 16
L = 16


def _sc_body(tid_hbm, w_hbm, b_hbm, out_hbm, idx_v, wrows_v, brows_v, eff_v,
             gsem, gsem2, ssem):
    wid = lax.axis_index("s") * SC_NC + lax.axis_index("c")

    pltpu.sync_copy(tid_hbm, idx_v)
    wg = pltpu.async_copy(w_hbm.at[idx_v], wrows_v, gsem)
    bg = pltpu.async_copy(b_hbm.at[idx_v], brows_v, gsem2)
    wg.wait()
    bg.wait()

    @pl.when(wid == 0)
    def _():
        is0 = idx_v[...] == 0
        for j in range(D // L):
            eff_v[pl.ds(L * j, L)] = (
                jnp.where(is0, 1.0, wrows_v[0, pl.ds(L * j, L)]))
            eff_v[pl.ds(D + L * j, L)] = (
                jnp.where(is0, 0.0, brows_v[0, pl.ds(L * j, L)]))
        pltpu.async_copy(eff_v, out_hbm, ssem).wait()


def _sc_lookup(tid_arr, weight, bias):
    mesh = plsc.VectorSubcoreMesh(core_axis_name="c", subcore_axis_name="s",
                                  num_cores=SC_NC, num_subcores=SC_NS)
    kern = pl.kernel(
        _sc_body,
        out_type=jax.ShapeDtypeStruct((2 * D,), jnp.float32),
        mesh=mesh,
        scratch_types=[
            pltpu.VMEM((L,), jnp.int32),
            pltpu.VMEM((L, D), jnp.float32),
            pltpu.VMEM((L, D), jnp.float32),
            pltpu.VMEM((2 * D,), jnp.float32),
            pltpu.SemaphoreType.DMA,
            pltpu.SemaphoreType.DMA,
            pltpu.SemaphoreType.DMA,
        ],
    )
    return kern(tid_arr, weight, bias)


TC_BLK = 8192


def _tc_body(x_ref, wb_ref, o_ref):
    w_eff = wb_ref[0:1, :]
    b_eff = wb_ref[1:2, :]
    o_ref[...] = x_ref[...] * w_eff + b_eff


def _tc_affine(x, wb):
    return pl.pallas_call(
        _tc_body,
        grid=(BATCH // TC_BLK,),
        in_specs=[
            pl.BlockSpec((TC_BLK, D), lambda i: (i, 0)),
            pl.BlockSpec((2, D), lambda i: (0, 0)),
        ],
        out_specs=pl.BlockSpec((TC_BLK, D), lambda i: (i, 0)),
        out_shape=jax.ShapeDtypeStruct((BATCH, D), jnp.float32),
        compiler_params=pltpu.CompilerParams(
            dimension_semantics=("parallel",)),
    )(x, wb)


@jax.jit
def _affine(x, tid_arr, weight, bias):
    wb = _sc_lookup(tid_arr, weight, bias).reshape(2, D)
    return _tc_affine(x, wb)


def kernel(x, task_id, weight, bias):
    tid_arr = jnp.full((L,), task_id, dtype=jnp.int32)
    return _affine(x, tid_arr, weight, bias)


# speedup vs baseline: 1.3394x; 1.3394x over previous
import jax
import jax.numpy as jnp
from jax import lax
from jax.experimental import pallas as pl
from jax.experimental.pallas import tpu as pltpu
from jax.experimental.pallas import tpu_sc as plsc

NB_TASKS = 1000
D = 128
BATCH = 16384

SC_NC = 1
SC_NS = 16
L = 16


def _sc_body(tid_hbm, w_hbm, b_hbm, out_hbm, idx_v, wrows_v, brows_v, eff_v,
             gsem, gsem2, ssem):
    wid = lax.axis_index("s") * SC_NC + lax.axis_index("c")

    @pl.when(wid == 0)
    def _():
        pltpu.sync_copy(tid_hbm, idx_v)
        wg = pltpu.async_copy(w_hbm.at[idx_v], wrows_v, gsem)
        bg = pltpu.async_copy(b_hbm.at[idx_v], brows_v, gsem2)
        wg.wait()
        bg.wait()

        is0 = idx_v[...] == 0
        for j in range(D // L):
            eff_v[pl.ds(L * j, L)] = (
                jnp.where(is0, 1.0, wrows_v[0, pl.ds(L * j, L)]))
            eff_v[pl.ds(D + L * j, L)] = (
                jnp.where(is0, 0.0, brows_v[0, pl.ds(L * j, L)]))
        pltpu.async_copy(eff_v, out_hbm, ssem).wait()


def _sc_lookup(tid_arr, weight, bias):
    mesh = plsc.VectorSubcoreMesh(core_axis_name="c", subcore_axis_name="s",
                                  num_cores=SC_NC, num_subcores=SC_NS)
    kern = pl.kernel(
        _sc_body,
        out_type=jax.ShapeDtypeStruct((2 * D,), jnp.float32),
        mesh=mesh,
        scratch_types=[
            pltpu.VMEM((L,), jnp.int32),
            pltpu.VMEM((L, D), jnp.float32),
            pltpu.VMEM((L, D), jnp.float32),
            pltpu.VMEM((2 * D,), jnp.float32),
            pltpu.SemaphoreType.DMA,
            pltpu.SemaphoreType.DMA,
            pltpu.SemaphoreType.DMA,
        ],
    )
    return kern(tid_arr, weight, bias)


TC_BLK = 8192


def _tc_body(x_ref, wb_ref, o_ref):
    w_eff = wb_ref[0:1, :]
    b_eff = wb_ref[1:2, :]
    o_ref[...] = x_ref[...] * w_eff + b_eff


def _tc_affine(x, wb):
    return pl.pallas_call(
        _tc_body,
        grid=(BATCH // TC_BLK,),
        in_specs=[
            pl.BlockSpec((TC_BLK, D), lambda i: (i, 0)),
            pl.BlockSpec((2, D), lambda i: (0, 0)),
        ],
        out_specs=pl.BlockSpec((TC_BLK, D), lambda i: (i, 0)),
        out_shape=jax.ShapeDtypeStruct((BATCH, D), jnp.float32),
        compiler_params=pltpu.CompilerParams(
            dimension_semantics=("parallel",)),
    )(x, wb)


@jax.jit
def _affine(x, tid_arr, weight, bias):
    wb = _sc_lookup(tid_arr, weight, bias).reshape(2, D)
    return _tc_affine(x, wb)


def kernel(x, task_id, weight, bias):
    tid_arr = jnp.full((L,), task_id, dtype=jnp.int32)
    return _affine(x, tid_arr, weight, bias)


# speedup vs baseline: 1.3395x; 1.0001x over previous
import jax
import jax.numpy as jnp
from jax import lax
from jax.experimental import pallas as pl
from jax.experimental.pallas import tpu as pltpu
from jax.experimental.pallas import tpu_sc as plsc

NB_TASKS = 1000
D = 128
BATCH = 16384

SC_NC = 1
SC_NS = 1
L = 16


def _sc_body(tid_hbm, w_hbm, b_hbm, out_hbm, idx_v, wrows_v, brows_v, eff_v,
             gsem, gsem2, ssem):
    wid = lax.axis_index("s") * SC_NC + lax.axis_index("c")

    @pl.when(wid == 0)
    def _():
        pltpu.sync_copy(tid_hbm, idx_v)
        wg = pltpu.async_copy(w_hbm.at[idx_v], wrows_v, gsem)
        bg = pltpu.async_copy(b_hbm.at[idx_v], brows_v, gsem2)
        wg.wait()
        bg.wait()

        is0 = idx_v[...] == 0
        for j in range(D // L):
            eff_v[pl.ds(L * j, L)] = (
                jnp.where(is0, 1.0, wrows_v[0, pl.ds(L * j, L)]))
            eff_v[pl.ds(D + L * j, L)] = (
                jnp.where(is0, 0.0, brows_v[0, pl.ds(L * j, L)]))
        pltpu.async_copy(eff_v, out_hbm, ssem).wait()


def _sc_lookup(tid_arr, weight, bias):
    mesh = plsc.VectorSubcoreMesh(core_axis_name="c", subcore_axis_name="s",
                                  num_cores=SC_NC, num_subcores=SC_NS)
    kern = pl.kernel(
        _sc_body,
        out_type=jax.ShapeDtypeStruct((2 * D,), jnp.float32),
        mesh=mesh,
        scratch_types=[
            pltpu.VMEM((L,), jnp.int32),
            pltpu.VMEM((L, D), jnp.float32),
            pltpu.VMEM((L, D), jnp.float32),
            pltpu.VMEM((2 * D,), jnp.float32),
            pltpu.SemaphoreType.DMA,
            pltpu.SemaphoreType.DMA,
            pltpu.SemaphoreType.DMA,
        ],
    )
    return kern(tid_arr, weight, bias)


TC_BLK = 8192


def _tc_body(x_ref, wb_ref, o_ref):
    w_eff = wb_ref[0:1, :]
    b_eff = wb_ref[1:2, :]
    o_ref[...] = x_ref[...] * w_eff + b_eff


def _tc_affine(x, wb):
    return pl.pallas_call(
        _tc_body,
        grid=(BATCH // TC_BLK,),
        in_specs=[
            pl.BlockSpec((TC_BLK, D), lambda i: (i, 0)),
            pl.BlockSpec((2, D), lambda i: (0, 0)),
        ],
        out_specs=pl.BlockSpec((TC_BLK, D), lambda i: (i, 0)),
        out_shape=jax.ShapeDtypeStruct((BATCH, D), jnp.float32),
        compiler_params=pltpu.CompilerParams(
            dimension_semantics=("parallel",)),
    )(x, wb)


@jax.jit
def _affine(x, tid_arr, weight, bias):
    wb = _sc_lookup(tid_arr, weight, bias).reshape(2, D)
    return _tc_affine(x, wb)


def kernel(x, task_id, weight, bias):
    tid_arr = jnp.full((L,), task_id, dtype=jnp.int32)
    return _affine(x, tid_arr, weight, bias)
